# baseline (speedup 1.0000x reference)
import functools

import jax
import jax.numpy as jnp
from jax import lax
from jax.experimental import pallas as pl
from jax.experimental.pallas import tpu as pltpu
from jax.experimental.pallas import tpu_sc as plsc

_N = 10000
_E = 320000
_G = 16

_B = 64
_K = 16
_NB = 5
_NSUB = 16
_EPAD = 327680
_IDXROWS = _EPAD // _B
_ACC_ROWS = 10008
_TRASH = 10000
_NPS = 624


def _make_agg(F, rows_per_tile, edge_split):
    mesh = plsc.VectorSubcoreMesh(core_axis_name="c", subcore_axis_name="s")
    nchunk = rows_per_tile // _K
    assert nchunk % 2 == 0
    out_t = (jax.ShapeDtypeStruct((_N, F), jnp.float32),
             jax.ShapeDtypeStruct((_N, F), jnp.float32))
    scratch = (
        [pltpu.VMEM((_K, _B), jnp.int32)] * 4 +
        [pltpu.VMEM((_B, F), jnp.float32)] * _NB +
        [pltpu.VMEM_SHARED((_ACC_ROWS, F), jnp.float32)] +
        [pltpu.SemaphoreType.DMA] * (2 * _NB + 2)
    )

    @functools.partial(pl.kernel, mesh=mesh, out_type=out_t,
                       scratch_types=scratch)
    def agg(src_hbm, dst_hbm, h0_hbm, h1_hbm, out0, out1, *refs):
        idx = [(refs[0], refs[1]), (refs[2], refs[3])]
        rows = list(refs[4:4 + _NB])
        acc = refs[4 + _NB]
        gsem = list(refs[5 + _NB:5 + 2 * _NB])
        ssem = list(refs[5 + 2 * _NB:5 + 3 * _NB])
        isem = list(refs[5 + 3 * _NB:7 + 3 * _NB])
        c = lax.axis_index("c")
        s = lax.axis_index("s")

        def rows_move(load):
            rbase = s * _NPS
            for i in range(4):
                load(pl.ds(rbase + i * 128, 128))
            load(pl.ds(rbase + 512, _NPS - 512))
            @pl.when(s == _NSUB - 1)
            def _():
                load(pl.ds(_NSUB * _NPS, _N - _NSUB * _NPS))

        def init_rows(h_hbm):
            rows_move(lambda sl: pltpu.sync_copy(h_hbm.at[sl], acc.at[sl]))

        @pl.when(c == 0)
        def _():
            init_rows(h0_hbm)

        @pl.when(c == 1)
        def _():
            init_rows(h1_hbm)

        plsc.subcore_barrier()

        def run_edges(h_hbm):
            if edge_split:
                row0 = c * (_IDXROWS // 2) + s * rows_per_tile
            else:
                row0 = s * rows_per_tile

            def load_idx(o, buf, sem):
                sl = pl.ds(row0 + o * _K, _K)
                return (pltpu.async_copy(src_hbm.at[sl], buf[0], sem),
                        pltpu.async_copy(dst_hbm.at[sl], buf[1], sem))

            def idx_wait(buf, sem):
                pltpu.make_async_copy(src_hbm.at[pl.ds(0, _K)], buf[0], sem).wait()
                pltpu.make_async_copy(dst_hbm.at[pl.ds(0, _K)], buf[1], sem).wait()

            def scatter_drain(p):
                pltpu.make_async_copy(
                    rows[p], acc.at[pl.ds(0, _B)], ssem[p]).wait()

            def chunk_body(o, o_is_first, cur, nxt, prefetch_pred):
                ci, cs = cur
                ni, ns = nxt
                if not o_is_first:
                    for p in range(_NB):
                        scatter_drain(p)
                    idx_wait(ci, cs)
                if isinstance(prefetch_pred, bool):
                    if prefetch_pred:
                        load_idx(o + 1, ni, ns)
                else:
                    @pl.when(prefetch_pred)
                    def _():
                        load_idx(o + 1, ni, ns)
                srcv, dstv = ci
                ghandle = [None] * _NB
                for p in range(_NB):
                    ghandle[p] = pltpu.async_copy(
                        h_hbm.at[srcv.at[p]], rows[p], gsem[p])
                for j in range(_K):
                    p = j % _NB
                    ghandle[p].wait()
                    sh = pltpu.async_copy(
                        rows[p], acc.at[dstv.at[j]], ssem[p], add=True)
                    if j + _NB < _K:
                        sh.wait()
                        ghandle[p] = pltpu.async_copy(
                            h_hbm.at[srcv.at[j + _NB]], rows[p], gsem[p])

            bufA = (idx[0], isem[0])
            bufB = (idx[1], isem[1])
            ha, hb = load_idx(0, idx[0], isem[0])
            ha.wait()
            hb.wait()

            def pair(o2, carry):
                o = 2 * o2
                chunk_body(o, False, bufA, bufB, o + 1 < nchunk)
                chunk_body(o + 1, False, bufB, bufA, o + 2 < nchunk)
                return carry

            chunk_body(0, True, bufA, bufB, True)
            chunk_body(1, False, bufB, bufA, nchunk > 2)
            lax.fori_loop(1, nchunk // 2, pair, 0)
            for p in range(_NB):
                scatter_drain(p)

        @pl.when(c == 0)
        def _():
            run_edges(h0_hbm)

        @pl.when(c == 1)
        def _():
            run_edges(h1_hbm)

        plsc.subcore_barrier()

        def copy_out(out_hbm):
            rows_move(lambda sl: pltpu.sync_copy(acc.at[sl], out_hbm.at[sl]))

        @pl.when(c == 0)
        def _():
            copy_out(out0)

        @pl.when(c == 1)
        def _():
            copy_out(out1)

    return agg


_agg_l0_inner = _make_agg(128, _IDXROWS // (2 * _NSUB), True)


def _dup_body(x_ref, o0_ref, o1_ref):
    o0_ref[...] = x_ref[...]
    o1_ref[...] = x_ref[...]


def _dup2(x):
    return pl.pallas_call(
        _dup_body,
        grid=(10,),
        in_specs=[pl.BlockSpec((1000, 128), lambda i: (i, 0))],
        out_specs=[pl.BlockSpec((1000, 128), lambda i: (i, 0)),
                   pl.BlockSpec((1000, 128), lambda i: (i, 0))],
        out_shape=[jax.ShapeDtypeStruct((_N, 128), jnp.float32),
                   jax.ShapeDtypeStruct((_N, 128), jnp.float32)],
    )(x)


def _agg_l0(src2d, dst2d, x):
    xa, xb = _dup2(x)
    return _agg_l0_inner(src2d, dst2d, xa, xb)


_agg128 = _make_agg(128, _IDXROWS // _NSUB, False)

_BLK = 1000
_NBLK = _N // _BLK


def _w_spec(shape):
    return pl.BlockSpec(shape, lambda i: (0,) * len(shape))


def _layer0_body(x_ref, a0_ref, a1_ref, w1_ref, b1_ref, w2_ref, b2_ref,
                 o0_ref, o1_ref):
    m = a0_ref[...] + a1_ref[...] - x_ref[...]
    t = jnp.maximum(
        jnp.dot(m, w1_ref[...], preferred_element_type=jnp.float32) + b1_ref[...], 0.0)
    h = jnp.maximum(
        jnp.dot(t, w2_ref[...], preferred_element_type=jnp.float32) + b2_ref[...], 0.0)
    o0_ref[...] = h[:, :128]
    o1_ref[...] = h[:, 128:]


def _layer0(x, a0, a1, w1, b1, w2, b2):
    return pl.pallas_call(
        _layer0_body,
        grid=(_NBLK,),
        in_specs=[
            pl.BlockSpec((_BLK, 128), lambda i: (i, 0)),
            pl.BlockSpec((_BLK, 128), lambda i: (i, 0)),
            pl.BlockSpec((_BLK, 128), lambda i: (i, 0)),
            _w_spec((128, 256)), _w_spec((1, 256)),
            _w_spec((256, 256)), _w_spec((1, 256)),
        ],
        out_specs=[pl.BlockSpec((_BLK, 128), lambda i: (i, 0)),
                   pl.BlockSpec((_BLK, 128), lambda i: (i, 0))],
        out_shape=[jax.ShapeDtypeStruct((_N, 128), jnp.float32),
                   jax.ShapeDtypeStruct((_N, 128), jnp.float32)],
    )(x, a0, a1, w1, b1, w2, b2)


def _layer_mid_body(m0_ref, m1_ref, w1_ref, b1_ref,
                    w2_ref, b2_ref, o0_ref, o1_ref):
    m = jnp.concatenate([m0_ref[...], m1_ref[...]], axis=1)
    t = jnp.maximum(
        jnp.dot(m, w1_ref[...], preferred_element_type=jnp.float32) + b1_ref[...], 0.0)
    h = jnp.maximum(
        jnp.dot(t, w2_ref[...], preferred_element_type=jnp.float32) + b2_ref[...], 0.0)
    o0_ref[...] = h[:, :128]
    o1_ref[...] = h[:, 128:]


def _layer_mid(m0, m1, w1, b1, w2, b2):
    return pl.pallas_call(
        _layer_mid_body,
        grid=(_NBLK,),
        in_specs=[
            pl.BlockSpec((_BLK, 128), lambda i: (i, 0)),
            pl.BlockSpec((_BLK, 128), lambda i: (i, 0)),
            _w_spec((256, 256)), _w_spec((1, 256)),
            _w_spec((256, 256)), _w_spec((1, 256)),
        ],
        out_specs=[pl.BlockSpec((_BLK, 128), lambda i: (i, 0)),
                   pl.BlockSpec((_BLK, 128), lambda i: (i, 0))],
        out_shape=[jax.ShapeDtypeStruct((_N, 128), jnp.float32),
                   jax.ShapeDtypeStruct((_N, 128), jnp.float32)],
    )(m0, m1, w1, b1, w2, b2)


def _layer_pool_body(m0_ref, m1_ref, w1_ref, b1_ref,
                     w2_ref, b2_ref, batch_ref, sums_ref, cnt_ref):
    i = pl.program_id(0)

    @pl.when(i == 0)
    def _():
        sums_ref[...] = jnp.zeros_like(sums_ref)
        cnt_ref[...] = jnp.zeros_like(cnt_ref)

    m = jnp.concatenate([m0_ref[...], m1_ref[...]], axis=1)
    t = jnp.maximum(
        jnp.dot(m, w1_ref[...], preferred_element_type=jnp.float32) + b1_ref[...], 0.0)
    h = jnp.maximum(
        jnp.dot(t, w2_ref[...], preferred_element_type=jnp.float32) + b2_ref[...], 0.0)

    b = jnp.broadcast_to(batch_ref[0], (_G, _BLK))
    gids = lax.broadcasted_iota(jnp.int32, (_G, _BLK), 0)
    onehot = (b == gids).astype(jnp.float32)
    sums_ref[...] += jnp.dot(onehot, h, preferred_element_type=jnp.float32)
    cnt_ref[...] += jnp.broadcast_to(
        jnp.sum(onehot, axis=1, keepdims=True), (_G, 128))


def _layer_pool(m0, m1, w1, b1, w2, b2, batch3d):
    return pl.pallas_call(
        _layer_pool_body,
        grid=(_NBLK,),
        in_specs=[
            pl.BlockSpec((_BLK, 128), lambda i: (i, 0)),
            pl.BlockSpec((_BLK, 128), lambda i: (i, 0)),
            _w_spec((256, 256)), _w_spec((1, 256)),
            _w_spec((256, 256)), _w_spec((1, 256)),
            pl.BlockSpec((1, 1, _BLK), lambda i: (i, 0, 0)),
        ],
        out_specs=[pl.BlockSpec((_G, 256), lambda i: (0, 0)),
                   pl.BlockSpec((_G, 128), lambda i: (0, 0))],
        out_shape=[jax.ShapeDtypeStruct((_G, 256), jnp.float32),
                   jax.ShapeDtypeStruct((_G, 128), jnp.float32)],
    )(m0, m1, w1, b1, w2, b2, batch3d)


def _head_body(sums_ref, cnt_ref, wh_ref, bh_ref, z_ref):
    g = sums_ref[...] / jnp.maximum(cnt_ref[:, :1], 1.0)
    z_ref[...] = jnp.dot(g, wh_ref[...], preferred_element_type=jnp.float32) + bh_ref[...]


def _head(sums, cnt, wh, bh):
    return pl.pallas_call(
        _head_body,
        in_specs=[pl.BlockSpec((_G, 256), lambda: (0, 0)),
                  pl.BlockSpec((_G, 128), lambda: (0, 0)),
                  pl.BlockSpec((256, 256), lambda: (0, 0)),
                  pl.BlockSpec((1, 256), lambda: (0, 0))],
        out_specs=pl.BlockSpec((_G, 256), lambda: (0, 0)),
        out_shape=jax.ShapeDtypeStruct((_G, 256), jnp.float32),
    )(sums, cnt, wh, bh)


def kernel(x, edge_index, batch,
           W1_0, b1_0, W2_0, b2_0,
           W1_1, b1_1, W2_1, b2_1,
           W1_2, b1_2, W2_2, b2_2,
           Wh, bh):
    pad = _EPAD - _E
    src2d = jnp.concatenate(
        [edge_index[0], jnp.zeros((pad,), jnp.int32)]).reshape(_IDXROWS, _B)
    dst2d = jnp.concatenate(
        [edge_index[1], jnp.full((pad,), _TRASH, jnp.int32)]).reshape(_IDXROWS, _B)
    batch3d = batch.reshape(_NBLK, 1, _BLK)
    b1_0r, b2_0r = b1_0.reshape(1, -1), b2_0.reshape(1, -1)
    b1_1r, b2_1r = b1_1.reshape(1, -1), b2_1.reshape(1, -1)
    b1_2r, b2_2r = b1_2.reshape(1, -1), b2_2.reshape(1, -1)
    bhr = bh.reshape(1, -1)

    p0, p1 = _agg_l0(src2d, dst2d, x)
    h1_lo, h1_hi = _layer0(x, p0, p1, W1_0, b1_0r, W2_0, b2_0r)
    m1_lo, m1_hi = _agg128(src2d, dst2d, h1_lo, h1_hi)
    h2_lo, h2_hi = _layer_mid(m1_lo, m1_hi, W1_1, b1_1r, W2_1, b2_1r)
    m2_lo, m2_hi = _agg128(src2d, dst2d, h2_lo, h2_hi)
    sums, cnt = _layer_pool(m2_lo, m2_hi, W1_2, b1_2r, W2_2, b2_2r, batch3d)
    return _head(sums, cnt, Wh, bhr)


# speedup vs baseline: 1.1698x; 1.1698x over previous
import functools

import jax
import jax.numpy as jnp
from jax import lax
from jax.experimental import pallas as pl
from jax.experimental.pallas import tpu as pltpu
from jax.experimental.pallas import tpu_sc as plsc

_N = 10000
_E = 320000
_G = 16

_B = 64
_K = 16
_NB = 5
_NSUB = 16
_EPAD = 327680
_IDXROWS = _EPAD // _B
_ACC_ROWS = 10008
_TRASH = 10000
_NPS = 624


def _make_agg(F, rows_per_tile, edge_split):
    mesh = plsc.VectorSubcoreMesh(core_axis_name="c", subcore_axis_name="s")
    nchunk = rows_per_tile // _K
    assert nchunk % 2 == 0
    out_t = (jax.ShapeDtypeStruct((_N, F), jnp.float32),
             jax.ShapeDtypeStruct((_N, F), jnp.float32))
    scratch = (
        [pltpu.VMEM((_K, _B), jnp.int32)] * 4 +
        [pltpu.VMEM((_B, F), jnp.float32)] * _NB +
        [pltpu.VMEM_SHARED((_ACC_ROWS, F), jnp.float32)] +
        [pltpu.SemaphoreType.DMA] * (2 * _NB + 2)
    )

    @functools.partial(pl.kernel, mesh=mesh, out_type=out_t,
                       scratch_types=scratch)
    def agg(src_hbm, dst_hbm, h0_hbm, h1_hbm, out0, out1, *refs):
        idx = [(refs[0], refs[1]), (refs[2], refs[3])]
        rows = list(refs[4:4 + _NB])
        acc = refs[4 + _NB]
        gsem = list(refs[5 + _NB:5 + 2 * _NB])
        ssem = list(refs[5 + 2 * _NB:5 + 3 * _NB])
        isem = list(refs[5 + 3 * _NB:7 + 3 * _NB])
        c = lax.axis_index("c")
        s = lax.axis_index("s")

        def rows_move(load):
            rbase = s * _NPS
            for i in range(4):
                load(pl.ds(rbase + i * 128, 128))
            load(pl.ds(rbase + 512, _NPS - 512))
            @pl.when(s == _NSUB - 1)
            def _():
                load(pl.ds(_NSUB * _NPS, _N - _NSUB * _NPS))

        def init_rows(h_hbm):
            rows_move(lambda sl: pltpu.sync_copy(h_hbm.at[sl], acc.at[sl]))

        @pl.when(c == 0)
        def _():
            init_rows(h0_hbm)

        @pl.when(c == 1)
        def _():
            init_rows(h1_hbm)

        plsc.subcore_barrier()

        def run_edges(h_hbm):
            if edge_split:
                row0 = c * (_IDXROWS // 2) + s * rows_per_tile
            else:
                row0 = s * rows_per_tile

            def load_idx(o, buf, sem):
                sl = pl.ds(row0 + o * _K, _K)
                return (pltpu.async_copy(src_hbm.at[sl], buf[0], sem),
                        pltpu.async_copy(dst_hbm.at[sl], buf[1], sem))

            def idx_wait(buf, sem):
                pltpu.make_async_copy(src_hbm.at[pl.ds(0, _K)], buf[0], sem).wait()
                pltpu.make_async_copy(dst_hbm.at[pl.ds(0, _K)], buf[1], sem).wait()

            def scatter_drain(p):
                pltpu.make_async_copy(
                    rows[p], acc.at[pl.ds(0, _B)], ssem[p]).wait()

            def chunk_body(o, o_is_first, cur, nxt, prefetch_pred):
                ci, cs = cur
                ni, ns = nxt
                if not o_is_first:
                    for p in range(_NB):
                        scatter_drain(p)
                    idx_wait(ci, cs)
                if isinstance(prefetch_pred, bool):
                    if prefetch_pred:
                        load_idx(o + 1, ni, ns)
                else:
                    @pl.when(prefetch_pred)
                    def _():
                        load_idx(o + 1, ni, ns)
                srcv, dstv = ci
                ghandle = [None] * _NB
                for p in range(_NB):
                    ghandle[p] = pltpu.async_copy(
                        h_hbm.at[srcv.at[p]], rows[p], gsem[p])
                for j in range(_K):
                    p = j % _NB
                    ghandle[p].wait()
                    sh = pltpu.async_copy(
                        rows[p], acc.at[dstv.at[j]], ssem[p], add=True)
                    if j + _NB < _K:
                        sh.wait()
                        ghandle[p] = pltpu.async_copy(
                            h_hbm.at[srcv.at[j + _NB]], rows[p], gsem[p])

            bufA = (idx[0], isem[0])
            bufB = (idx[1], isem[1])
            ha, hb = load_idx(0, idx[0], isem[0])
            ha.wait()
            hb.wait()

            def pair(o2, carry):
                o = 2 * o2
                chunk_body(o, False, bufA, bufB, o + 1 < nchunk)
                chunk_body(o + 1, False, bufB, bufA, o + 2 < nchunk)
                return carry

            chunk_body(0, True, bufA, bufB, True)
            chunk_body(1, False, bufB, bufA, nchunk > 2)
            lax.fori_loop(1, nchunk // 2, pair, 0)
            for p in range(_NB):
                scatter_drain(p)

        @pl.when(c == 0)
        def _():
            run_edges(h0_hbm)

        @pl.when(c == 1)
        def _():
            run_edges(h1_hbm)

        plsc.subcore_barrier()

        def copy_out(out_hbm):
            rows_move(lambda sl: pltpu.sync_copy(acc.at[sl], out_hbm.at[sl]))

        @pl.when(c == 0)
        def _():
            copy_out(out0)

        @pl.when(c == 1)
        def _():
            copy_out(out1)

    return agg


_agg_l0_inner = _make_agg(128, _IDXROWS // (2 * _NSUB), True)


def _dup_body(x_ref, o_ref):
    o_ref[...] = x_ref[...]


def _dup(x):
    return pl.pallas_call(
        _dup_body,
        grid=(10,),
        in_specs=[pl.BlockSpec((1000, 128), lambda i: (i, 0))],
        out_specs=pl.BlockSpec((1000, 128), lambda i: (i, 0)),
        out_shape=jax.ShapeDtypeStruct((_N, 128), jnp.float32),
    )(x)


def _agg_l0(src2d, dst2d, x):
    return _agg_l0_inner(src2d, dst2d, x, _dup(x))


_agg128 = _make_agg(128, _IDXROWS // _NSUB, False)

_BLK = 1000
_NBLK = _N // _BLK


def _w_spec(shape):
    return pl.BlockSpec(shape, lambda i: (0,) * len(shape))


def _layer0_body(x_ref, a0_ref, a1_ref, w1_ref, b1_ref, w2_ref, b2_ref,
                 o0_ref, o1_ref):
    m = a0_ref[...] + a1_ref[...] - x_ref[...]
    t = jnp.maximum(
        jnp.dot(m, w1_ref[...], preferred_element_type=jnp.float32) + b1_ref[...], 0.0)
    h = jnp.maximum(
        jnp.dot(t, w2_ref[...], preferred_element_type=jnp.float32) + b2_ref[...], 0.0)
    o0_ref[...] = h[:, :128]
    o1_ref[...] = h[:, 128:]


def _layer0(x, a0, a1, w1, b1, w2, b2):
    return pl.pallas_call(
        _layer0_body,
        grid=(_NBLK,),
        in_specs=[
            pl.BlockSpec((_BLK, 128), lambda i: (i, 0)),
            pl.BlockSpec((_BLK, 128), lambda i: (i, 0)),
            pl.BlockSpec((_BLK, 128), lambda i: (i, 0)),
            _w_spec((128, 256)), _w_spec((1, 256)),
            _w_spec((256, 256)), _w_spec((1, 256)),
        ],
        out_specs=[pl.BlockSpec((_BLK, 128), lambda i: (i, 0)),
                   pl.BlockSpec((_BLK, 128), lambda i: (i, 0))],
        out_shape=[jax.ShapeDtypeStruct((_N, 128), jnp.float32),
                   jax.ShapeDtypeStruct((_N, 128), jnp.float32)],
    )(x, a0, a1, w1, b1, w2, b2)


def _layer_mid_body(m0_ref, m1_ref, w1_ref, b1_ref,
                    w2_ref, b2_ref, o0_ref, o1_ref):
    m = jnp.concatenate([m0_ref[...], m1_ref[...]], axis=1)
    t = jnp.maximum(
        jnp.dot(m, w1_ref[...], preferred_element_type=jnp.float32) + b1_ref[...], 0.0)
    h = jnp.maximum(
        jnp.dot(t, w2_ref[...], preferred_element_type=jnp.float32) + b2_ref[...], 0.0)
    o0_ref[...] = h[:, :128]
    o1_ref[...] = h[:, 128:]


def _layer_mid(m0, m1, w1, b1, w2, b2):
    return pl.pallas_call(
        _layer_mid_body,
        grid=(_NBLK,),
        in_specs=[
            pl.BlockSpec((_BLK, 128), lambda i: (i, 0)),
            pl.BlockSpec((_BLK, 128), lambda i: (i, 0)),
            _w_spec((256, 256)), _w_spec((1, 256)),
            _w_spec((256, 256)), _w_spec((1, 256)),
        ],
        out_specs=[pl.BlockSpec((_BLK, 128), lambda i: (i, 0)),
                   pl.BlockSpec((_BLK, 128), lambda i: (i, 0))],
        out_shape=[jax.ShapeDtypeStruct((_N, 128), jnp.float32),
                   jax.ShapeDtypeStruct((_N, 128), jnp.float32)],
    )(m0, m1, w1, b1, w2, b2)


def _layer_pool_body(m0_ref, m1_ref, w1_ref, b1_ref,
                     w2_ref, b2_ref, batch_ref, sums_ref, cnt_ref):
    i = pl.program_id(0)

    @pl.when(i == 0)
    def _():
        sums_ref[...] = jnp.zeros_like(sums_ref)
        cnt_ref[...] = jnp.zeros_like(cnt_ref)

    m = jnp.concatenate([m0_ref[...], m1_ref[...]], axis=1)
    t = jnp.maximum(
        jnp.dot(m, w1_ref[...], preferred_element_type=jnp.float32) + b1_ref[...], 0.0)
    h = jnp.maximum(
        jnp.dot(t, w2_ref[...], preferred_element_type=jnp.float32) + b2_ref[...], 0.0)

    b = jnp.broadcast_to(batch_ref[0], (_G, _BLK))
    gids = lax.broadcasted_iota(jnp.int32, (_G, _BLK), 0)
    onehot = (b == gids).astype(jnp.float32)
    sums_ref[...] += jnp.dot(onehot, h, preferred_element_type=jnp.float32)
    cnt_ref[...] += jnp.broadcast_to(
        jnp.sum(onehot, axis=1, keepdims=True), (_G, 128))


def _layer_pool(m0, m1, w1, b1, w2, b2, batch3d):
    return pl.pallas_call(
        _layer_pool_body,
        grid=(_NBLK,),
        in_specs=[
            pl.BlockSpec((_BLK, 128), lambda i: (i, 0)),
            pl.BlockSpec((_BLK, 128), lambda i: (i, 0)),
            _w_spec((256, 256)), _w_spec((1, 256)),
            _w_spec((256, 256)), _w_spec((1, 256)),
            pl.BlockSpec((1, 1, _BLK), lambda i: (i, 0, 0)),
        ],
        out_specs=[pl.BlockSpec((_G, 256), lambda i: (0, 0)),
                   pl.BlockSpec((_G, 128), lambda i: (0, 0))],
        out_shape=[jax.ShapeDtypeStruct((_G, 256), jnp.float32),
                   jax.ShapeDtypeStruct((_G, 128), jnp.float32)],
    )(m0, m1, w1, b1, w2, b2, batch3d)


def _head_body(sums_ref, cnt_ref, wh_ref, bh_ref, z_ref):
    g = sums_ref[...] / jnp.maximum(cnt_ref[:, :1], 1.0)
    z_ref[...] = jnp.dot(g, wh_ref[...], preferred_element_type=jnp.float32) + bh_ref[...]


def _head(sums, cnt, wh, bh):
    return pl.pallas_call(
        _head_body,
        in_specs=[pl.BlockSpec((_G, 256), lambda: (0, 0)),
                  pl.BlockSpec((_G, 128), lambda: (0, 0)),
                  pl.BlockSpec((256, 256), lambda: (0, 0)),
                  pl.BlockSpec((1, 256), lambda: (0, 0))],
        out_specs=pl.BlockSpec((_G, 256), lambda: (0, 0)),
        out_shape=jax.ShapeDtypeStruct((_G, 256), jnp.float32),
    )(sums, cnt, wh, bh)


def kernel(x, edge_index, batch,
           W1_0, b1_0, W2_0, b2_0,
           W1_1, b1_1, W2_1, b2_1,
           W1_2, b1_2, W2_2, b2_2,
           Wh, bh):
    pad = _EPAD - _E
    src2d = jnp.concatenate(
        [edge_index[0], jnp.zeros((pad,), jnp.int32)]).reshape(_IDXROWS, _B)
    dst2d = jnp.concatenate(
        [edge_index[1], jnp.full((pad,), _TRASH, jnp.int32)]).reshape(_IDXROWS, _B)
    batch3d = batch.reshape(_NBLK, 1, _BLK)
    b1_0r, b2_0r = b1_0.reshape(1, -1), b2_0.reshape(1, -1)
    b1_1r, b2_1r = b1_1.reshape(1, -1), b2_1.reshape(1, -1)
    b1_2r, b2_2r = b1_2.reshape(1, -1), b2_2.reshape(1, -1)
    bhr = bh.reshape(1, -1)

    p0, p1 = _agg_l0(src2d, dst2d, x)
    h1_lo, h1_hi = _layer0(x, p0, p1, W1_0, b1_0r, W2_0, b2_0r)
    m1_lo, m1_hi = _agg128(src2d, dst2d, h1_lo, h1_hi)
    h2_lo, h2_hi = _layer_mid(m1_lo, m1_hi, W1_1, b1_1r, W2_1, b2_1r)
    m2_lo, m2_hi = _agg128(src2d, dst2d, h2_lo, h2_hi)
    sums, cnt = _layer_pool(m2_lo, m2_hi, W1_2, b1_2r, W2_2, b2_2r, batch3d)
    return _head(sums, cnt, Wh, bhr)


# speedup vs baseline: 1.1847x; 1.0128x over previous
import functools

import jax
import jax.numpy as jnp
from jax import lax
from jax.experimental import pallas as pl
from jax.experimental.pallas import tpu as pltpu
from jax.experimental.pallas import tpu_sc as plsc

_N = 10000
_E = 320000
_G = 16

_B = 64
_K = 16
_NB = 5
_NSUB = 16
_EPAD = 327680
_IDXROWS = _EPAD // _B
_ACC_ROWS = 10008
_TRASH = 10000
_NPS = 624


def _make_agg(F, rows_per_tile, edge_split, K=_K, NB=_NB):
    mesh = plsc.VectorSubcoreMesh(core_axis_name="c", subcore_axis_name="s")
    nchunk = rows_per_tile // K
    assert nchunk % 2 == 0
    out_t = (jax.ShapeDtypeStruct((_N, F), jnp.float32),
             jax.ShapeDtypeStruct((_N, F), jnp.float32))
    scratch = (
        [pltpu.VMEM((K, _B), jnp.int32)] * 4 +
        [pltpu.VMEM((_B, F), jnp.float32)] * NB +
        [pltpu.VMEM_SHARED((_ACC_ROWS, F), jnp.float32)] +
        [pltpu.SemaphoreType.DMA] * (2 * NB + 2)
    )

    @functools.partial(pl.kernel, mesh=mesh, out_type=out_t,
                       scratch_types=scratch)
    def agg(src_hbm, dst_hbm, h0_hbm, h1_hbm, out0, out1, *refs):
        idx = [(refs[0], refs[1]), (refs[2], refs[3])]
        rows = list(refs[4:4 + NB])
        acc = refs[4 + NB]
        gsem = list(refs[5 + NB:5 + 2 * NB])
        ssem = list(refs[5 + 2 * NB:5 + 3 * NB])
        isem = list(refs[5 + 3 * NB:7 + 3 * NB])
        c = lax.axis_index("c")
        s = lax.axis_index("s")

        def rows_move(load):
            rbase = s * _NPS
            for i in range(4):
                load(pl.ds(rbase + i * 128, 128))
            load(pl.ds(rbase + 512, _NPS - 512))
            @pl.when(s == _NSUB - 1)
            def _():
                load(pl.ds(_NSUB * _NPS, _N - _NSUB * _NPS))

        def init_rows(h_hbm):
            rows_move(lambda sl: pltpu.sync_copy(h_hbm.at[sl], acc.at[sl]))

        @pl.when(c == 0)
        def _():
            init_rows(h0_hbm)

        @pl.when(c == 1)
        def _():
            init_rows(h1_hbm)

        plsc.subcore_barrier()

        def run_edges(h_hbm):
            if edge_split:
                row0 = c * (_IDXROWS // 2) + s * rows_per_tile
            else:
                row0 = s * rows_per_tile

            def load_idx(o, buf, sem):
                sl = pl.ds(row0 + o * K, K)
                return (pltpu.async_copy(src_hbm.at[sl], buf[0], sem),
                        pltpu.async_copy(dst_hbm.at[sl], buf[1], sem))

            def idx_wait(buf, sem):
                pltpu.make_async_copy(src_hbm.at[pl.ds(0, K)], buf[0], sem).wait()
                pltpu.make_async_copy(dst_hbm.at[pl.ds(0, K)], buf[1], sem).wait()

            def scatter_drain(p):
                pltpu.make_async_copy(
                    rows[p], acc.at[pl.ds(0, _B)], ssem[p]).wait()

            def chunk_body(o, o_is_first, cur, nxt, prefetch_pred):
                ci, cs = cur
                ni, ns = nxt
                if not o_is_first:
                    for p in range(NB):
                        scatter_drain(p)
                    idx_wait(ci, cs)
                if isinstance(prefetch_pred, bool):
                    if prefetch_pred:
                        load_idx(o + 1, ni, ns)
                else:
                    @pl.when(prefetch_pred)
                    def _():
                        load_idx(o + 1, ni, ns)
                srcv, dstv = ci
                ghandle = [None] * NB
                for p in range(NB):
                    ghandle[p] = pltpu.async_copy(
                        h_hbm.at[srcv.at[p]], rows[p], gsem[p])
                for j in range(K):
                    p = j % NB
                    ghandle[p].wait()
                    sh = pltpu.async_copy(
                        rows[p], acc.at[dstv.at[j]], ssem[p], add=True)
                    if j + NB < K:
                        sh.wait()
                        ghandle[p] = pltpu.async_copy(
                            h_hbm.at[srcv.at[j + NB]], rows[p], gsem[p])

            bufA = (idx[0], isem[0])
            bufB = (idx[1], isem[1])
            ha, hb = load_idx(0, idx[0], isem[0])
            ha.wait()
            hb.wait()

            def pair(o2, carry):
                o = 2 * o2
                chunk_body(o, False, bufA, bufB, o + 1 < nchunk)
                chunk_body(o + 1, False, bufB, bufA, o + 2 < nchunk)
                return carry

            chunk_body(0, True, bufA, bufB, True)
            chunk_body(1, False, bufB, bufA, nchunk > 2)
            lax.fori_loop(1, nchunk // 2, pair, 0)
            for p in range(NB):
                scatter_drain(p)

        @pl.when(c == 0)
        def _():
            run_edges(h0_hbm)

        @pl.when(c == 1)
        def _():
            run_edges(h1_hbm)

        plsc.subcore_barrier()

        def copy_out(out_hbm):
            rows_move(lambda sl: pltpu.sync_copy(acc.at[sl], out_hbm.at[sl]))

        @pl.when(c == 0)
        def _():
            copy_out(out0)

        @pl.when(c == 1)
        def _():
            copy_out(out1)

    return agg


_agg_l0_inner = _make_agg(128, _IDXROWS // (2 * _NSUB), True)


def _dup_body(x_ref, o_ref):
    o_ref[...] = x_ref[...]


def _dup(x):
    return pl.pallas_call(
        _dup_body,
        grid=(10,),
        in_specs=[pl.BlockSpec((1000, 128), lambda i: (i, 0))],
        out_specs=pl.BlockSpec((1000, 128), lambda i: (i, 0)),
        out_shape=jax.ShapeDtypeStruct((_N, 128), jnp.float32),
    )(x)


def _agg_l0(src2d, dst2d, x):
    return _agg_l0_inner(src2d, dst2d, x, _dup(x))


_agg128 = _make_agg(128, _IDXROWS // _NSUB, False, K=32, NB=4)

_BLK = 1000
_NBLK = _N // _BLK


def _w_spec(shape):
    return pl.BlockSpec(shape, lambda i: (0,) * len(shape))


def _layer0_body(x_ref, a0_ref, a1_ref, w1_ref, b1_ref, w2_ref, b2_ref,
                 o0_ref, o1_ref):
    m = a0_ref[...] + a1_ref[...] - x_ref[...]
    t = jnp.maximum(
        jnp.dot(m, w1_ref[...], preferred_element_type=jnp.float32) + b1_ref[...], 0.0)
    h = jnp.maximum(
        jnp.dot(t, w2_ref[...], preferred_element_type=jnp.float32) + b2_ref[...], 0.0)
    o0_ref[...] = h[:, :128]
    o1_ref[...] = h[:, 128:]


def _layer0(x, a0, a1, w1, b1, w2, b2):
    return pl.pallas_call(
        _layer0_body,
        grid=(_NBLK,),
        in_specs=[
            pl.BlockSpec((_BLK, 128), lambda i: (i, 0)),
            pl.BlockSpec((_BLK, 128), lambda i: (i, 0)),
            pl.BlockSpec((_BLK, 128), lambda i: (i, 0)),
            _w_spec((128, 256)), _w_spec((1, 256)),
            _w_spec((256, 256)), _w_spec((1, 256)),
        ],
        out_specs=[pl.BlockSpec((_BLK, 128), lambda i: (i, 0)),
                   pl.BlockSpec((_BLK, 128), lambda i: (i, 0))],
        out_shape=[jax.ShapeDtypeStruct((_N, 128), jnp.float32),
                   jax.ShapeDtypeStruct((_N, 128), jnp.float32)],
    )(x, a0, a1, w1, b1, w2, b2)


def _layer_mid_body(m0_ref, m1_ref, w1_ref, b1_ref,
                    w2_ref, b2_ref, o0_ref, o1_ref):
    m = jnp.concatenate([m0_ref[...], m1_ref[...]], axis=1)
    t = jnp.maximum(
        jnp.dot(m, w1_ref[...], preferred_element_type=jnp.float32) + b1_ref[...], 0.0)
    h = jnp.maximum(
        jnp.dot(t, w2_ref[...], preferred_element_type=jnp.float32) + b2_ref[...], 0.0)
    o0_ref[...] = h[:, :128]
    o1_ref[...] = h[:, 128:]


def _layer_mid(m0, m1, w1, b1, w2, b2):
    return pl.pallas_call(
        _layer_mid_body,
        grid=(_NBLK,),
        in_specs=[
            pl.BlockSpec((_BLK, 128), lambda i: (i, 0)),
            pl.BlockSpec((_BLK, 128), lambda i: (i, 0)),
            _w_spec((256, 256)), _w_spec((1, 256)),
            _w_spec((256, 256)), _w_spec((1, 256)),
        ],
        out_specs=[pl.BlockSpec((_BLK, 128), lambda i: (i, 0)),
                   pl.BlockSpec((_BLK, 128), lambda i: (i, 0))],
        out_shape=[jax.ShapeDtypeStruct((_N, 128), jnp.float32),
                   jax.ShapeDtypeStruct((_N, 128), jnp.float32)],
    )(m0, m1, w1, b1, w2, b2)


def _layer_pool_body(m0_ref, m1_ref, w1_ref, b1_ref,
                     w2_ref, b2_ref, batch_ref, sums_ref, cnt_ref):
    i = pl.program_id(0)

    @pl.when(i == 0)
    def _():
        sums_ref[...] = jnp.zeros_like(sums_ref)
        cnt_ref[...] = jnp.zeros_like(cnt_ref)

    m = jnp.concatenate([m0_ref[...], m1_ref[...]], axis=1)
    t = jnp.maximum(
        jnp.dot(m, w1_ref[...], preferred_element_type=jnp.float32) + b1_ref[...], 0.0)
    h = jnp.maximum(
        jnp.dot(t, w2_ref[...], preferred_element_type=jnp.float32) + b2_ref[...], 0.0)

    b = jnp.broadcast_to(batch_ref[0], (_G, _BLK))
    gids = lax.broadcasted_iota(jnp.int32, (_G, _BLK), 0)
    onehot = (b == gids).astype(jnp.float32)
    sums_ref[...] += jnp.dot(onehot, h, preferred_element_type=jnp.float32)
    cnt_ref[...] += jnp.broadcast_to(
        jnp.sum(onehot, axis=1, keepdims=True), (_G, 128))


def _layer_pool(m0, m1, w1, b1, w2, b2, batch3d):
    return pl.pallas_call(
        _layer_pool_body,
        grid=(_NBLK,),
        in_specs=[
            pl.BlockSpec((_BLK, 128), lambda i: (i, 0)),
            pl.BlockSpec((_BLK, 128), lambda i: (i, 0)),
            _w_spec((256, 256)), _w_spec((1, 256)),
            _w_spec((256, 256)), _w_spec((1, 256)),
            pl.BlockSpec((1, 1, _BLK), lambda i: (i, 0, 0)),
        ],
        out_specs=[pl.BlockSpec((_G, 256), lambda i: (0, 0)),
                   pl.BlockSpec((_G, 128), lambda i: (0, 0))],
        out_shape=[jax.ShapeDtypeStruct((_G, 256), jnp.float32),
                   jax.ShapeDtypeStruct((_G, 128), jnp.float32)],
    )(m0, m1, w1, b1, w2, b2, batch3d)


def _head_body(sums_ref, cnt_ref, wh_ref, bh_ref, z_ref):
    g = sums_ref[...] / jnp.maximum(cnt_ref[:, :1], 1.0)
    z_ref[...] = jnp.dot(g, wh_ref[...], preferred_element_type=jnp.float32) + bh_ref[...]


def _head(sums, cnt, wh, bh):
    return pl.pallas_call(
        _head_body,
        in_specs=[pl.BlockSpec((_G, 256), lambda: (0, 0)),
                  pl.BlockSpec((_G, 128), lambda: (0, 0)),
                  pl.BlockSpec((256, 256), lambda: (0, 0)),
                  pl.BlockSpec((1, 256), lambda: (0, 0))],
        out_specs=pl.BlockSpec((_G, 256), lambda: (0, 0)),
        out_shape=jax.ShapeDtypeStruct((_G, 256), jnp.float32),
    )(sums, cnt, wh, bh)


def kernel(x, edge_index, batch,
           W1_0, b1_0, W2_0, b2_0,
           W1_1, b1_1, W2_1, b2_1,
           W1_2, b1_2, W2_2, b2_2,
           Wh, bh):
    pad = _EPAD - _E
    src2d = jnp.concatenate(
        [edge_index[0], jnp.zeros((pad,), jnp.int32)]).reshape(_IDXROWS, _B)
    dst2d = jnp.concatenate(
        [edge_index[1], jnp.full((pad,), _TRASH, jnp.int32)]).reshape(_IDXROWS, _B)
    batch3d = batch.reshape(_NBLK, 1, _BLK)
    b1_0r, b2_0r = b1_0.reshape(1, -1), b2_0.reshape(1, -1)
    b1_1r, b2_1r = b1_1.reshape(1, -1), b2_1.reshape(1, -1)
    b1_2r, b2_2r = b1_2.reshape(1, -1), b2_2.reshape(1, -1)
    bhr = bh.reshape(1, -1)

    p0, p1 = _agg_l0(src2d, dst2d, x)
    h1_lo, h1_hi = _layer0(x, p0, p1, W1_0, b1_0r, W2_0, b2_0r)
    m1_lo, m1_hi = _agg128(src2d, dst2d, h1_lo, h1_hi)
    h2_lo, h2_hi = _layer_mid(m1_lo, m1_hi, W1_1, b1_1r, W2_1, b2_1r)
    m2_lo, m2_hi = _agg128(src2d, dst2d, h2_lo, h2_hi)
    sums, cnt = _layer_pool(m2_lo, m2_hi, W1_2, b1_2r, W2_2, b2_2r, batch3d)
    return _head(sums, cnt, Wh, bhr)
